# Initial kernel scaffold; baseline (speedup 1.0000x reference)
import jax
import jax.numpy as jnp
from jax.experimental import pallas as pl


def kernel(h_init, x_i, edge_index_i, node_ids_i, x_j, edge_index_j, node_ids_j, W_gat, attn_l, attn_r, b_gat, W_out, b_out):
    raise NotImplementedError("write your pallas kernel here")


# speedup vs baseline: 34.2373x; 34.2373x over previous
import functools

import jax
import jax.numpy as jnp
from jax import lax
from jax.experimental import pallas as pl
from jax.experimental.pallas import tpu as pltpu
from jax.experimental.pallas import tpu_sc as plsc

N = 10000
D = 128
NH = 8
NC, NS = 2, 16

EC = 128
CHUNKS = 157
EPT = EC * CHUNKS
E_PAD = NS * EPT
RSH = 10240
TRASH = 10200

NP = 10240
PT = 10496
SENT = 10240


def _prologue_body(x_ref, w_ref, a_ref, feat_ref, lg_ref):
    feat = jnp.dot(x_ref[:], w_ref[:], preferred_element_type=jnp.float32)
    feat_ref[:] = feat
    lg_ref[:] = jnp.dot(feat, a_ref[:], preferred_element_type=jnp.float32)


def _tc_prologue(x_both, w_gat, a_pad):
    m = x_both.shape[0]
    br = 1000
    return pl.pallas_call(
        _prologue_body,
        grid=(m // br,),
        in_specs=[
            pl.BlockSpec((br, D), lambda i: (i, 0)),
            pl.BlockSpec((D, D), lambda i: (0, 0)),
            pl.BlockSpec((D, D), lambda i: (0, 0)),
        ],
        out_specs=[
            pl.BlockSpec((br, D), lambda i: (i, 0)),
            pl.BlockSpec((br, D), lambda i: (i, 0)),
        ],
        out_shape=[
            jax.ShapeDtypeStruct((m, D), jnp.float32),
            jax.ShapeDtypeStruct((m, D), jnp.float32),
        ],
    )(x_both, w_gat, a_pad)


def _epilogue_body(u_ref, s_ref, r_ref, w_ref, bg_ref, p_ref):
    s_full = jnp.dot(s_ref[:], r_ref[:], preferred_element_type=jnp.float32)
    u = u_ref[:]
    g = jnp.where(s_full > 0.0, u / s_full, 0.0) + bg_ref[:]
    nrm = jnp.sqrt(jnp.sum(g * g, axis=1, keepdims=True))
    gn = g / jnp.maximum(nrm, 1e-12)
    p_ref[:] = jnp.dot(gn, w_ref[0], preferred_element_type=jnp.float32)


def _tc_epilogue(u2, s2, r_m, w_both, bg):
    m = u2.shape[0]
    br = 1000
    nb = m // br
    return pl.pallas_call(
        _epilogue_body,
        grid=(nb,),
        in_specs=[
            pl.BlockSpec((br, D), lambda i: (i, 0)),
            pl.BlockSpec((br, D), lambda i: (i, 0)),
            pl.BlockSpec((D, D), lambda i: (0, 0)),
            pl.BlockSpec((1, D, D), lambda i: (i // (nb // 2), 0, 0)),
            pl.BlockSpec((1, D), lambda i: (0, 0)),
        ],
        out_specs=pl.BlockSpec((br, D), lambda i: (i, 0)),
        out_shape=jax.ShapeDtypeStruct((m, D), jnp.float32),
    )(u2, s2, r_m, w_both, bg)


@functools.cache
def _get_ex_kernel():
  return pl.kernel(
    _ex_kernel,
    out_type=jax.ShapeDtypeStruct((NC * E_PAD * 16,), jnp.float32),
    mesh=plsc.VectorSubcoreMesh(core_axis_name="c", subcore_axis_name="s", num_cores=NC, num_subcores=NS),
    scratch_types=[
        pltpu.VMEM((EC, D), jnp.float32),
        pltpu.VMEM((EC, D), jnp.float32),
        pltpu.VMEM((EC * 16,), jnp.float32),
        pltpu.VMEM((EC,), jnp.int32),
        pltpu.VMEM((EC,), jnp.int32),
        pltpu.SemaphoreType.DMA,
    ])


def _ex_kernel(lg_hbm, srcg_hbm, dstg_hbm, ex_hbm,
               lgs_buf, lgd_buf, ex_buf, srcg_v, dstg_v, sem):
    c = lax.axis_index("c")
    t = lax.axis_index("s")
    tbase = c * E_PAD + t * EPT

    def chunk(kk, _):
        base = tbase + kk * EC
        pltpu.sync_copy(srcg_hbm.at[pl.ds(base, EC)], srcg_v)
        pltpu.sync_copy(dstg_hbm.at[pl.ds(base, EC)], dstg_v)
        pltpu.async_copy(lg_hbm.at[srcg_v], lgs_buf, sem).wait()
        pltpu.async_copy(lg_hbm.at[dstg_v], lgd_buf, sem).wait()

        def edge(i, _):
            ev = lgs_buf[i, pl.ds(0, 16)] + lgd_buf[i, pl.ds(16, 16)]
            ev = jnp.where(ev > 0.0, ev, ev * jnp.float32(0.2))
            ex_buf[pl.ds(16 * i, 16)] = jnp.exp(ev)
            return 0

        lax.fori_loop(0, EC, edge, 0)
        pltpu.sync_copy(ex_buf, ex_hbm.at[pl.ds(base * 16, EC * 16)])
        return 0

    lax.fori_loop(0, CHUNKS, chunk, 0)


@functools.cache
def _get_s_kernel():
  return pl.kernel(
    _s_kernel,
    out_type=jax.ShapeDtypeStruct((NC, N, D), jnp.float32),
    mesh=plsc.VectorSubcoreMesh(core_axis_name="c", subcore_axis_name="s", num_cores=NC, num_subcores=NS),
    scratch_types=[
        pltpu.VMEM_SHARED((RSH, D), jnp.float32),
        pltpu.VMEM((EC, D), jnp.float32),
        pltpu.VMEM((EC * 16,), jnp.float32),
        pltpu.VMEM((EC,), jnp.int32),
        pltpu.SemaphoreType.DMA,
    ])


def _s_kernel(ex_hbm, dstl_hbm, s_hbm, s_sh, exw_buf, ex_buf, dstl_v, sem):
    c = lax.axis_index("c")
    t = lax.axis_index("s")
    zf = jnp.zeros((16,), jnp.float32)

    def zrow(i, _):
        for h in range(NH):
            exw_buf[i, pl.ds(16 * h, 16)] = zf
        return 0

    lax.fori_loop(0, EC, zrow, 0)
    z0 = t * (RSH // NS)
    for k in range(RSH // NS // EC):
        pltpu.sync_copy(exw_buf, s_sh.at[pl.ds(z0 + k * EC, EC)])
    plsc.subcore_barrier()

    tbase = c * E_PAD + t * EPT

    def chunk(kk, _):
        base = tbase + kk * EC
        pltpu.sync_copy(dstl_hbm.at[pl.ds(base, EC)], dstl_v)
        pltpu.sync_copy(ex_hbm.at[pl.ds(base * 16, EC * 16)], ex_buf)

        def edge(i, _):
            exw_buf[i, pl.ds(0, 16)] = ex_buf[pl.ds(16 * i, 16)]
            return 0

        lax.fori_loop(0, EC, edge, 0)
        pltpu.sync_copy(exw_buf, s_sh.at[dstl_v], add=True)
        return 0

    lax.fori_loop(0, CHUNKS, chunk, 0)
    plsc.subcore_barrier()
    nr = (N // NS) // 8 * 8
    o0 = t * nr
    pltpu.sync_copy(s_sh.at[pl.ds(o0, nr)], s_hbm.at[c, pl.ds(o0, nr)])

    @pl.when(t == 0)
    def _tail():
        o1 = NS * nr
        pltpu.sync_copy(s_sh.at[pl.ds(o1, N - o1)], s_hbm.at[c, pl.ds(o1, N - o1)])


@functools.cache
def _get_message_kernel():
  return pl.kernel(
    _message_kernel,
    out_type=jax.ShapeDtypeStruct((NC, N, D), jnp.float32),
    mesh=plsc.VectorSubcoreMesh(core_axis_name="c", subcore_axis_name="s", num_cores=NC, num_subcores=NS),
    scratch_types=[
        pltpu.VMEM_SHARED((RSH, D), jnp.float32),
        pltpu.VMEM((EC, D), jnp.float32),
        pltpu.VMEM((EC * 16,), jnp.float32),
        pltpu.VMEM((EC,), jnp.int32),
        pltpu.VMEM((EC,), jnp.int32),
        pltpu.SemaphoreType.DMA,
    ])


def _message_kernel(feat_hbm, ex_hbm, srcg_hbm, dstl_hbm, u_hbm,
                    u_sh, feat_buf, ex_buf, srcg_v, dstl_v, sem):
    c = lax.axis_index("c")
    t = lax.axis_index("s")
    zf = jnp.zeros((16,), jnp.float32)

    def zrow(i, _):
        for h in range(NH):
            feat_buf[i, pl.ds(16 * h, 16)] = zf
        return 0

    lax.fori_loop(0, EC, zrow, 0)
    z0 = t * (RSH // NS)
    for k in range(RSH // NS // EC):
        pltpu.sync_copy(feat_buf, u_sh.at[pl.ds(z0 + k * EC, EC)])
    plsc.subcore_barrier()

    tbase = c * E_PAD + t * EPT

    def chunk(kk, _):
        base = tbase + kk * EC
        pltpu.sync_copy(srcg_hbm.at[pl.ds(base, EC)], srcg_v)
        pltpu.sync_copy(dstl_hbm.at[pl.ds(base, EC)], dstl_v)
        pltpu.sync_copy(ex_hbm.at[pl.ds(base * 16, EC * 16)], ex_buf)
        pltpu.async_copy(feat_hbm.at[srcg_v], feat_buf, sem).wait()

        def edge(i, _):
            exv = ex_buf[pl.ds(16 * i, 16)]
            for h in range(NH):
                sl = pl.ds(16 * h, 16)
                feat_buf[i, sl] = feat_buf[i, sl] * exv[h]
            return 0

        lax.fori_loop(0, EC, edge, 0)
        pltpu.sync_copy(feat_buf, u_sh.at[dstl_v], add=True)
        return 0

    lax.fori_loop(0, CHUNKS, chunk, 0)
    plsc.subcore_barrier()
    nr = (N // NS) // 8 * 8
    o0 = t * nr
    pltpu.sync_copy(u_sh.at[pl.ds(o0, nr)], u_hbm.at[c, pl.ds(o0, nr)])

    @pl.when(t == 0)
    def _tail():
        o1 = NS * nr
        pltpu.sync_copy(u_sh.at[pl.ds(o1, N - o1)], u_hbm.at[c, pl.ds(o1, N - o1)])


@functools.cache
def _get_presence_kernel():
  return pl.kernel(
    _presence_kernel,
    out_type=jax.ShapeDtypeStruct((NC * NP,), jnp.int32),
    mesh=plsc.VectorSubcoreMesh(core_axis_name="c", subcore_axis_name="s", num_cores=NC, num_subcores=NS),
    scratch_types=[
        pltpu.VMEM_SHARED((PT,), jnp.int32),
        pltpu.VMEM_SHARED((PT,), jnp.int32),
        pltpu.VMEM((PT // NS,), jnp.int32),
        pltpu.VMEM((NP // NS // 128, 128), jnp.int32),
        pltpu.VMEM((128,), jnp.int32),
    ])


def _presence_kernel(idsp_hbm, pres_hbm, pres_a, pres_b, zbuf, ids2d,
                     ones_buf):
    c = lax.axis_index("c")
    t = lax.axis_index("s")
    zi = jnp.zeros((16,), jnp.int32)
    ones = jnp.ones((16,), jnp.int32)
    slc = PT // NS

    def zscan(i, _):
        zbuf[pl.ds(16 * i, 16)] = zi
        return 0

    lax.fori_loop(0, slc // 16, zscan, 0)

    def fill_ones(i, _):
        ones_buf[pl.ds(16 * i, 16)] = ones
        return 0

    lax.fori_loop(0, 8, fill_ones, 0)
    pltpu.sync_copy(zbuf, pres_a.at[pl.ds(t * slc, slc)])
    pltpu.sync_copy(zbuf, pres_b.at[pl.ds(t * slc, slc)])
    plsc.subcore_barrier()

    def scatter_graph(g, pres_sh):
        pltpu.sync_copy(idsp_hbm.at[g, t], ids2d)
        for j in range(NP // NS // 128):
            pltpu.sync_copy(ones_buf, pres_sh.at[ids2d.at[j]])

    scatter_graph(0, pres_a)
    scatter_graph(1, pres_b)
    plsc.subcore_barrier()
    o0 = t * (NP // NS)

    @pl.when(c == 0)
    def _copy_a():
        pltpu.sync_copy(pres_a.at[pl.ds(o0, NP // NS)], pres_hbm.at[pl.ds(o0, NP // NS)])

    @pl.when(c == 1)
    def _copy_b():
        pltpu.sync_copy(pres_b.at[pl.ds(o0, NP // NS)], pres_hbm.at[pl.ds(NP + o0, NP // NS)])


def _prefix_body(p_ref, tl_ref, tr_ref, pref_ref):
    p = p_ref[:].astype(jnp.float32)
    p = jnp.minimum(p, 1.0)
    within = jnp.dot(p, tl_ref[:], preferred_element_type=jnp.float32)
    rowsum = jnp.sum(p, axis=1, keepdims=True)
    offs = jnp.dot(tr_ref[:], rowsum, preferred_element_type=jnp.float32)
    pref_ref[:] = (within + offs).astype(jnp.int32)


def _tc_prefix(pres2d, t_lane, t_row):
    m = pres2d.shape[0]
    return pl.pallas_call(
        _prefix_body,
        grid=(1,),
        in_specs=[
            pl.BlockSpec((m, 128), lambda i: (0, 0)),
            pl.BlockSpec((128, 128), lambda i: (0, 0)),
            pl.BlockSpec((m, m), lambda i: (0, 0)),
        ],
        out_specs=pl.BlockSpec((m, 128), lambda i: (0, 0)),
        out_shape=jax.ShapeDtypeStruct((m, 128), jnp.int32),
    )(pres2d, t_lane, t_row)


@functools.cache
def _get_final_kernel():
  return pl.kernel(
    _final_kernel,
    out_type=jax.ShapeDtypeStruct((NP, D), jnp.float32),
    mesh=plsc.VectorSubcoreMesh(core_axis_name="c", subcore_axis_name="s", num_cores=NC, num_subcores=NS),
    scratch_types=[
        pltpu.VMEM_SHARED((NP,), jnp.int32),
        pltpu.VMEM_SHARED((NP,), jnp.int32),
        pltpu.VMEM((64,), jnp.int32),
        pltpu.VMEM((64,), jnp.int32),
        pltpu.VMEM((64,), jnp.int32),
        pltpu.VMEM((64,), jnp.int32),
        pltpu.VMEM((64, D), jnp.float32),
        pltpu.VMEM((64, D), jnp.float32),
        pltpu.VMEM((D,), jnp.float32),
        pltpu.SemaphoreType.DMA,
    ])


def _final_kernel(pref_hbm, idsr_hbm, p0_hbm, p1_hbm, bout_hbm, out_hbm,
                  loc_a, loc_b, ida, idb, rka, rkb, rows_a, rows_b, bout_v,
                  sem):
    c = lax.axis_index("c")
    t = lax.axis_index("s")
    slc = NP // NS
    pltpu.sync_copy(pref_hbm.at[pl.ds(t * slc, slc)], loc_a.at[pl.ds(t * slc, slc)])
    pltpu.sync_copy(pref_hbm.at[pl.ds(NP + t * slc, slc)], loc_b.at[pl.ds(t * slc, slc)])
    pltpu.sync_copy(bout_hbm, bout_v)
    plsc.subcore_barrier()
    w32 = t * NC + c
    base_row = w32 * (NP // (NC * NS))

    def rowchunk(k, _):
        r0 = base_row + k * 64
        pltpu.sync_copy(idsr_hbm.at[pl.ds(r0, 64)], ida)
        pltpu.sync_copy(idsr_hbm.at[pl.ds(NP + r0, 64)], idb)
        pltpu.async_copy(loc_a.at[ida], rka, sem).wait()
        pltpu.async_copy(loc_b.at[idb], rkb, sem).wait()
        pltpu.async_copy(p0_hbm.at[rka], rows_a, sem).wait()
        pltpu.async_copy(p1_hbm.at[rkb], rows_b, sem).wait()

        def comb(r, _):
            for h in range(NH):
                sl = pl.ds(16 * h, 16)
                rows_a[r, sl] = rows_a[r, sl] + rows_b[r, sl] + bout_v[sl]
            return 0

        lax.fori_loop(0, 64, comb, 0)
        pltpu.sync_copy(rows_a, out_hbm.at[pl.ds(r0, 64)])
        return 0

    lax.fori_loop(0, NP // (NC * NS) // 64, rowchunk, 0)


def kernel(h_init, x_i, edge_index_i, node_ids_i, x_j, edge_index_j,
           node_ids_j, W_gat, attn_l, attn_r, b_gat, W_out, b_out):
    f32 = jnp.float32
    x_both = jnp.concatenate([x_i, x_j], axis=0)
    eye8 = jnp.eye(NH, dtype=f32)
    a_l = (attn_l[:, :, None] * eye8[:, None, :]).reshape(D, NH)
    a_r = (attn_r[:, :, None] * eye8[:, None, :]).reshape(D, NH)
    a_big = jnp.concatenate(
        [a_l, a_r, a_r, a_l, jnp.zeros((D, D - 32), f32)], axis=1)
    feat, lg = _tc_prologue(x_both, W_gat, a_big)

    e = edge_index_i.shape[1]
    pad = E_PAD - e
    zpad = jnp.zeros((pad,), jnp.int32)
    tpad = jnp.full((pad,), TRASH, jnp.int32)
    srcg = jnp.concatenate([edge_index_i[0], zpad,
                            edge_index_j[0] + N, zpad])
    dstg = jnp.concatenate([edge_index_i[1], zpad,
                            edge_index_j[1] + N, zpad])
    dstl = jnp.concatenate([edge_index_i[1], tpad,
                            edge_index_j[1], tpad])
    ex_all = _get_ex_kernel()(lg, srcg, dstg)
    s = _get_s_kernel()(ex_all, dstl)
    u = _get_message_kernel()(feat, ex_all, srcg, dstl)

    r_m = (eye8[:, :, None] * jnp.ones((1, 1, 16), f32)).reshape(NH, D)
    r_m = jnp.concatenate([r_m, jnp.zeros((D - NH, D), f32)], axis=0)
    w_both = jnp.stack([W_out[:D], W_out[D:]])
    p = _tc_epilogue(u.reshape(NC * N, D), s.reshape(NC * N, D), r_m,
                     w_both, b_gat.reshape(1, D))
    p0, p1 = p[:N], p[N:]

    spad = jnp.full((NP - N,), SENT, jnp.int32)
    znp = jnp.zeros((NP - N,), jnp.int32)
    idsp = jnp.stack([
        jnp.concatenate([node_ids_i, spad]),
        jnp.concatenate([node_ids_j, spad]),
    ]).reshape(2, NS, NP // NS // 128, 128)
    idsr = jnp.concatenate([node_ids_i, znp, node_ids_j, znp])
    pres = _get_presence_kernel()(idsp)
    t_lane = jnp.triu(jnp.ones((128, 128), f32), 1)
    t_row = jnp.kron(jnp.eye(2, dtype=f32),
                     jnp.tril(jnp.ones((NP // 128, NP // 128), f32), -1))
    pref = _tc_prefix(pres.reshape(NC * NP // 128, 128), t_lane, t_row)
    hout = _get_final_kernel()(pref.reshape(NC * NP), idsr, p0, p1, b_out)
    return jnp.stack([hout[:N]])
